# baseline (speedup 1.0000x reference)
import jax
import jax.numpy as jnp
from jax import lax
from jax.experimental import pallas as pl
from jax.experimental.pallas import tpu as pltpu
from jax.experimental.pallas import tpu_sc as plsc

_FIELD_DIM = 100000
_B = 16384
_F = 26
_TOTAL = _FIELD_DIM * _F

_NC = 2
_NS = 16
_NW = _NC * _NS
_BPW = _B // _NW
_EPW = _BPW * _F
_LANES = 16

_CHUNK = 128
_NCHUNK = _EPW // _CHUNK
_DEPTH = 16

_QW = 81152
_TAIL_OFF = _NW * _QW
_TAIL = _TOTAL - _TAIL_OFF
_QH = _QW // 2


def _prep_body(tab2_hbm, x_hbm, flat_hbm, idx_hbm,
               buf0, buf1, tbuf, xv, idxv, sem, tsem, xsem):
    wid = lax.axis_index("s") * _NC + lax.axis_index("c")
    base = wid * _QW
    bbase = wid * _BPW

    r0 = pltpu.async_copy(tab2_hbm.at[0, pl.ds(base, _QH)], buf0, sem)
    r1 = pltpu.async_copy(tab2_hbm.at[0, pl.ds(base + _QH, _QH)], buf1, tsem)

    xcps = [pltpu.async_copy(x_hbm.at[pl.ds(f * _B + bbase, _BPW)],
                             xv.at[pl.ds(f * _BPW, _BPW)], xsem)
            for f in range(_F)]

    @pl.when(wid == 0)
    def _():
        pltpu.sync_copy(tab2_hbm.at[0, pl.ds(_TAIL_OFF, _TAIL)], tbuf)

    for cp in xcps:
        cp.wait()

    def idx_group(j, _):
        o = j * _LANES
        for f in range(_F):
            off = jnp.int32(f * _FIELD_DIM)
            idxv[pl.ds(f * _BPW + o, _LANES)] = (
                xv[pl.ds(f * _BPW + o, _LANES)] + off
            )
        return 0
    lax.fori_loop(0, _BPW // _LANES, idx_group, 0)
    wx = pltpu.async_copy(idxv, idx_hbm.at[pl.ds(wid * _EPW, _EPW)], xsem)

    r0.wait()
    w0 = pltpu.async_copy(buf0, flat_hbm.at[pl.ds(base, _QH)], sem)
    r1.wait()
    w1 = pltpu.async_copy(buf1, flat_hbm.at[pl.ds(base + _QH, _QH)], tsem)

    @pl.when(wid == 0)
    def _():
        pltpu.sync_copy(tbuf, flat_hbm.at[pl.ds(_TAIL_OFF, _TAIL)])

    w0.wait()
    w1.wait()
    wx.wait()


def _lookup_body(idx_hbm, w_hbm, table_hbm, out_hbm,
                 idxv, wv, embv, outv, sem, wsem):
    wid = lax.axis_index("s") * _NC + lax.axis_index("c")
    bbase = wid * _BPW

    cp_i = pltpu.async_copy(idx_hbm.at[pl.ds(wid * _EPW, _EPW)], idxv, wsem)
    wcps = [pltpu.async_copy(w_hbm.at[pl.ds(f * _B + bbase, _BPW)],
                             wv.at[pl.ds(f * _BPW, _BPW)], wsem)
            for f in range(_F)]
    cp_i.wait()

    def fire(off):
        return pltpu.async_copy(
            table_hbm.at[idxv.at[pl.ds(off, _CHUNK)]],
            embv.at[pl.ds(off, _CHUNK)], sem)

    def wait_one():
        pltpu.make_async_copy(
            table_hbm.at[idxv.at[pl.ds(0, _CHUNK)]],
            embv.at[pl.ds(0, _CHUNK)], sem).wait()

    for c in range(_DEPTH):
        fire(c * _CHUNK)

    def gather_step(c, _):
        wait_one()
        fire(c * _CHUNK)
        return 0
    lax.fori_loop(_DEPTH, _NCHUNK, gather_step, 0)
    for cp in wcps:
        cp.wait()
    for _ in range(_DEPTH):
        wait_one()

    def reduce_group(g, _):
        rbase = g * _LANES
        acc = jnp.zeros((_LANES,), jnp.float32)
        for f in range(_F):
            o = f * _BPW + rbase
            acc = acc + embv[pl.ds(o, _LANES)] * wv[pl.ds(o, _LANES)]
        outv[pl.ds(rbase, _LANES)] = acc
        return 0
    lax.fori_loop(0, _BPW // _LANES, reduce_group, 0)

    pltpu.sync_copy(outv, out_hbm.at[pl.ds(bbase, _BPW)])


_MESH = plsc.VectorSubcoreMesh(core_axis_name="c", subcore_axis_name="s")


@jax.jit
def _sc_prep(table2d, x_t):
    f = pl.kernel(
        _prep_body,
        out_type=(jax.ShapeDtypeStruct((_TOTAL,), jnp.float32),
                  jax.ShapeDtypeStruct((_B * _F,), jnp.int32)),
        mesh=_MESH,
        scratch_types=[
            pltpu.VMEM((_QH,), jnp.float32),
            pltpu.VMEM((_QH,), jnp.float32),
            pltpu.VMEM((_TAIL,), jnp.float32),
            pltpu.VMEM((_EPW,), jnp.int32),
            pltpu.VMEM((_EPW,), jnp.int32),
            pltpu.SemaphoreType.DMA,
            pltpu.SemaphoreType.DMA,
            pltpu.SemaphoreType.DMA,
        ],
    )
    return f(table2d, x_t)


@jax.jit
def _sc_lookup(idx_flat, w_t, table):
    f = pl.kernel(
        _lookup_body,
        out_type=jax.ShapeDtypeStruct((_B,), jnp.float32),
        mesh=_MESH,
        scratch_types=[
            pltpu.VMEM((_EPW,), jnp.int32),
            pltpu.VMEM((_EPW,), jnp.float32),
            pltpu.VMEM((_EPW,), jnp.float32),
            pltpu.VMEM((_BPW,), jnp.float32),
            pltpu.SemaphoreType.DMA,
            pltpu.SemaphoreType.DMA,
        ],
    )
    return f(idx_flat, w_t, table)


def kernel(x, weight, fc_table, bias):
    x_t = x.T.reshape(-1)
    w_t = jnp.transpose(weight, (1, 2, 0)).reshape(-1)
    table, idx_flat = _sc_prep(fc_table.T, x_t)
    out = _sc_lookup(idx_flat, w_t, table)
    return out[:, None] + bias[None, :]


# speedup vs baseline: 1.2287x; 1.2287x over previous
import jax
import jax.numpy as jnp
from jax import lax
from jax.experimental import pallas as pl
from jax.experimental.pallas import tpu as pltpu
from jax.experimental.pallas import tpu_sc as plsc

_FIELD_DIM = 100000
_B = 16384
_F = 26
_TOTAL = _FIELD_DIM * _F

_NC = 2
_NS = 16
_NW = _NC * _NS
_BPW = _B // _NW
_EPW = _BPW * _F
_LANES = 16

_CHUNK = 128
_NCHUNK = _EPW // _CHUNK
_DEPTH = 16


def _body(tab2_hbm, x_hbm, w_hbm, out_hbm,
          xv, wv, idxv, embv, outv, sem, wsem):
    wid = lax.axis_index("s") * _NC + lax.axis_index("c")
    bbase = wid * _BPW

    xcps = [pltpu.async_copy(x_hbm.at[pl.ds(f * _B + bbase, _BPW)],
                             xv.at[pl.ds(f * _BPW, _BPW)], sem)
            for f in range(_F)]
    wcps = [pltpu.async_copy(w_hbm.at[pl.ds(f * _B + bbase, _BPW)],
                             wv.at[pl.ds(f * _BPW, _BPW)], wsem)
            for f in range(_F)]
    for cp in xcps:
        cp.wait()

    def idx_group(j, _):
        o = j * _LANES
        for f in range(_F):
            off = jnp.int32(f * _FIELD_DIM)
            idxv[pl.ds(f * _BPW + o, _LANES)] = (
                xv[pl.ds(f * _BPW + o, _LANES)] + off
            )
        return 0
    lax.fori_loop(0, _BPW // _LANES, idx_group, 0)

    t1d = tab2_hbm.at[0]

    def fire(off):
        return pltpu.async_copy(
            t1d.at[idxv.at[pl.ds(off, _CHUNK)]],
            embv.at[pl.ds(off, _CHUNK)], sem)

    def wait_one():
        pltpu.make_async_copy(
            t1d.at[idxv.at[pl.ds(0, _CHUNK)]],
            embv.at[pl.ds(0, _CHUNK)], sem).wait()

    for c in range(_DEPTH):
        fire(c * _CHUNK)

    def gather_step(c, _):
        wait_one()
        fire(c * _CHUNK)
        return 0
    lax.fori_loop(_DEPTH, _NCHUNK, gather_step, 0)
    for cp in wcps:
        cp.wait()
    for _ in range(_DEPTH):
        wait_one()

    def reduce_group(g, _):
        rbase = g * _LANES
        acc = jnp.zeros((_LANES,), jnp.float32)
        for f in range(_F):
            o = f * _BPW + rbase
            acc = acc + embv[pl.ds(o, _LANES)] * wv[pl.ds(o, _LANES)]
        outv[pl.ds(rbase, _LANES)] = acc
        return 0
    lax.fori_loop(0, _BPW // _LANES, reduce_group, 0)

    pltpu.sync_copy(outv, out_hbm.at[pl.ds(bbase, _BPW)])


_MESH = plsc.VectorSubcoreMesh(core_axis_name="c", subcore_axis_name="s")


@jax.jit
def _sc_lookup(table2d, x_t, w_t):
    f = pl.kernel(
        _body,
        out_type=jax.ShapeDtypeStruct((_B,), jnp.float32),
        mesh=_MESH,
        scratch_types=[
            pltpu.VMEM((_EPW,), jnp.int32),
            pltpu.VMEM((_EPW,), jnp.float32),
            pltpu.VMEM((_EPW,), jnp.int32),
            pltpu.VMEM((_EPW,), jnp.float32),
            pltpu.VMEM((_BPW,), jnp.float32),
            pltpu.SemaphoreType.DMA,
            pltpu.SemaphoreType.DMA,
        ],
    )
    return f(table2d, x_t, w_t)


def kernel(x, weight, fc_table, bias):
    x_t = x.T.reshape(-1)
    w_t = jnp.transpose(weight, (1, 2, 0)).reshape(-1)
    out = _sc_lookup(fc_table.T, x_t, w_t)
    return out[:, None] + bias[None, :]
